# baseline (speedup 1.0000x reference)
import functools

import jax
import jax.numpy as jnp
from jax import lax
from jax.experimental import pallas as pl
from jax.experimental.pallas import tpu as pltpu
from jax.experimental.pallas import tpu_sc as plsc

N_EVENTS_TOTAL = 64
N_RES = 2048
N_SAMPLES = 16384

NC, NS, L = 2, 16, 16
ROWS_PER_W = N_EVENTS_TOTAL // (NC * NS)


def _lane_perm(x, perm):
    dn = lax.GatherDimensionNumbers(
        offset_dims=(), collapsed_slice_dims=(0,), start_index_map=(0,))
    return lax.gather(x, perm[:, None], dn, slice_sizes=(1,),
                      mode=lax.GatherScatterMode.PROMISE_IN_BOUNDS)


def _sc_select_gather(sel_hbm, items_hbm, out_hbm, sel_v, rows_v, sem):
    wid = lax.axis_index("s") * NC + lax.axis_index("c")
    base = wid * ROWS_PER_W

    pltpu.sync_copy(sel_hbm.at[pl.ds(base, ROWS_PER_W)], sel_v)

    lanes = lax.iota(jnp.int32, L)

    def row_argmax(r):
        def body(j, carry):
            bv, bi = carry
            v = sel_v[r, pl.ds(j * L, L)]
            idxs = j * L + lanes
            take = v > bv
            return jnp.where(take, v, bv), jnp.where(take, idxs, bi)

        bv, bi = lax.fori_loop(
            0, N_RES // L, body,
            (jnp.full((L,), -jnp.inf, jnp.float32),
             jnp.zeros((L,), jnp.int32)))
        for shift in (8, 4, 2, 1):
            perm = lanes ^ shift
            ov = _lane_perm(bv, perm)
            oi = _lane_perm(bi, perm)
            better = (ov > bv) | ((ov == bv) & (oi < bi))
            bv = jnp.where(better, ov, bv)
            bi = jnp.where(better, oi, bi)
        return bi

    idx0 = row_argmax(0)
    idx1 = row_argmax(1)
    iv = jnp.where(lanes == 1, idx1, idx0)

    pltpu.async_copy(items_hbm.at[iv], rows_v, sem).wait()
    pltpu.sync_copy(rows_v.at[pl.ds(0, ROWS_PER_W)],
                    out_hbm.at[pl.ds(base, ROWS_PER_W)])


def _select_gather(sel2d, items):
    mesh = plsc.VectorSubcoreMesh(core_axis_name="c", subcore_axis_name="s")
    return pl.kernel(
        _sc_select_gather,
        mesh=mesh,
        out_type=jax.ShapeDtypeStruct((N_EVENTS_TOTAL, N_RES), jnp.float32),
        scratch_types=[
            pltpu.VMEM((ROWS_PER_W, N_RES), jnp.float32),
            pltpu.VMEM((L, N_RES), jnp.float32),
            pltpu.SemaphoreType.DMA,
        ],
    )(sel2d, items)


KB = 256


def _mm_body(g_ref, w_ref, o_ref):
    k = pl.program_id(0)
    part = jnp.dot(g_ref[...].astype(jnp.bfloat16),
                   w_ref[...].astype(jnp.bfloat16),
                   preferred_element_type=jnp.float32)

    @pl.when(k == 0)
    def _init():
        o_ref[...] = part

    @pl.when(k != 0)
    def _acc():
        o_ref[...] += part


def _postprocess(gathered, waves):
    return pl.pallas_call(
        _mm_body,
        grid=(N_RES // KB,),
        in_specs=[
            pl.BlockSpec((N_EVENTS_TOTAL, KB), lambda k: (0, k)),
            pl.BlockSpec((KB, N_SAMPLES), lambda k: (k, 0)),
        ],
        out_specs=pl.BlockSpec((N_EVENTS_TOTAL, N_SAMPLES), lambda k: (0, 0)),
        out_shape=jax.ShapeDtypeStruct((N_EVENTS_TOTAL, N_SAMPLES),
                                       jnp.float32),
    )(gathered, waves)


def kernel(selections, items, waves):
    b, e, x, n = selections.shape
    sel2d = selections.reshape(b * e * x, n)
    gathered = _select_gather(sel2d, items)
    out = _postprocess(gathered, waves)
    return out.reshape(b, e, x, N_SAMPLES)


# speedup vs baseline: 1.2097x; 1.2097x over previous
import functools

import jax
import jax.numpy as jnp
from jax import lax
from jax.experimental import pallas as pl
from jax.experimental.pallas import tpu as pltpu
from jax.experimental.pallas import tpu_sc as plsc

N_EVENTS = 16
INSTR = 4
N_ROWS = 64
N_RES = 2048
N_SAMPLES = 16384

NC, NS, L = 2, 16, 16
ROWS_PER_W = N_ROWS // (NC * NS)


def _lane_perm(x, perm):
    dn = lax.GatherDimensionNumbers(
        offset_dims=(), collapsed_slice_dims=(0,), start_index_map=(0,))
    return lax.gather(x, perm[:, None], dn, slice_sizes=(1,),
                      mode=lax.GatherScatterMode.PROMISE_IN_BOUNDS)


def _sc_select_gather(sel_hbm, items_hbm, out_hbm, sel_v, idx_v, rows_v, sem):
    wid = lax.axis_index("s") * NC + lax.axis_index("c")
    base = wid * ROWS_PER_W

    pltpu.sync_copy(sel_hbm.at[pl.ds(base, ROWS_PER_W)], sel_v)

    lanes = lax.iota(jnp.int32, L)
    ninf = jnp.full((L,), -jnp.inf, jnp.float32)
    zero = jnp.zeros((L,), jnp.int32)

    def body(j, carry):
        bv0, bi0, bv1, bi1 = carry
        b = j * (4 * L)
        for u in range(4):
            off = b + u * L
            idxs = off + lanes
            v0 = sel_v[0, pl.ds(off, L)]
            v1 = sel_v[1, pl.ds(off, L)]
            t0 = v0 > bv0
            t1 = v1 > bv1
            bv0 = jnp.where(t0, v0, bv0)
            bi0 = jnp.where(t0, idxs, bi0)
            bv1 = jnp.where(t1, v1, bv1)
            bi1 = jnp.where(t1, idxs, bi1)
        return bv0, bi0, bv1, bi1

    bv0, bi0, bv1, bi1 = lax.fori_loop(
        0, N_RES // (4 * L), body, (ninf, zero, ninf, zero))

    for shift in (8, 4, 2, 1):
        perm = lanes ^ shift
        ov0, oi0 = _lane_perm(bv0, perm), _lane_perm(bi0, perm)
        ov1, oi1 = _lane_perm(bv1, perm), _lane_perm(bi1, perm)
        b0 = (ov0 > bv0) | ((ov0 == bv0) & (oi0 < bi0))
        b1 = (ov1 > bv1) | ((ov1 == bv1) & (oi1 < bi1))
        bv0 = jnp.where(b0, ov0, bv0)
        bi0 = jnp.where(b0, oi0, bi0)
        bv1 = jnp.where(b1, ov1, bv1)
        bi1 = jnp.where(b1, oi1, bi1)

    iv = jnp.where(lanes == 1, bi1, bi0)
    idx_v[...] = iv

    pltpu.async_copy(items_hbm.at[idx_v.at[pl.ds(0, ROWS_PER_W)]],
                     rows_v, sem).wait()
    pltpu.sync_copy(rows_v, out_hbm.at[pl.ds(base, ROWS_PER_W)])


def _select_gather(sel2d, items):
    mesh = plsc.VectorSubcoreMesh(core_axis_name="c", subcore_axis_name="s")
    return pl.kernel(
        _sc_select_gather,
        mesh=mesh,
        out_type=jax.ShapeDtypeStruct((N_ROWS, N_RES), jnp.float32),
        scratch_types=[
            pltpu.VMEM((ROWS_PER_W, N_RES), jnp.float32),
            pltpu.VMEM((L,), jnp.int32),
            pltpu.VMEM((ROWS_PER_W, N_RES), jnp.float32),
            pltpu.SemaphoreType.DMA,
        ],
    )(sel2d, items)


NT = 1024


def _mm_body(g_ref, w_ref, o_ref):
    part = jnp.dot(g_ref[...], w_ref[...],
                   preferred_element_type=jnp.float32)
    for e in range(N_EVENTS):
        o_ref[0, e] = lax.slice_in_dim(part, INSTR * e, INSTR * (e + 1),
                                       axis=0)


def _postprocess(gathered, waves):
    return pl.pallas_call(
        _mm_body,
        grid=(N_SAMPLES // NT,),
        in_specs=[
            pl.BlockSpec((N_ROWS, N_RES), lambda j: (0, 0)),
            pl.BlockSpec((N_RES, NT), lambda j: (0, j)),
        ],
        out_specs=pl.BlockSpec((1, N_EVENTS, INSTR, NT),
                               lambda j: (0, 0, 0, j)),
        out_shape=jax.ShapeDtypeStruct((1, N_EVENTS, INSTR, N_SAMPLES),
                                       jnp.float32),
    )(gathered, waves)


def kernel(selections, items, waves):
    b, e, x, n = selections.shape
    sel2d = selections.reshape(b * e * x, n)
    gathered = _select_gather(sel2d, items)
    return _postprocess(gathered, waves)


# speedup vs baseline: 1.2188x; 1.0075x over previous
import jax
import jax.numpy as jnp
from jax import lax
from jax.experimental import pallas as pl
from jax.experimental.pallas import tpu as pltpu
from jax.experimental.pallas import tpu_sc as plsc

N_EVENTS = 16
INSTR = 4
N_ROWS = 64
N_RES = 2048
N_SAMPLES = 16384

NC, NS, L = 2, 16, 16
ROWS_PER_W = N_ROWS // (NC * NS)


def _lane_perm(x, perm):
    dn = lax.GatherDimensionNumbers(
        offset_dims=(), collapsed_slice_dims=(0,), start_index_map=(0,))
    return lax.gather(x, perm[:, None], dn, slice_sizes=(1,),
                      mode=lax.GatherScatterMode.PROMISE_IN_BOUNDS)


def _sc_select_gather(sel_hbm, items_hbm, out_hbm, sel_v, idx_v, rows_v, sem):
    wid = lax.axis_index("s") * NC + lax.axis_index("c")
    base = wid * ROWS_PER_W

    pltpu.sync_copy(sel_hbm.at[pl.ds(base, ROWS_PER_W)], sel_v)

    lanes = lax.iota(jnp.int32, L)
    ninf = jnp.full((L,), -jnp.inf, jnp.float32)
    zero = jnp.zeros((L,), jnp.int32)

    def body(j, carry):
        bv0, bi0, bv1, bi1 = carry
        b = j * (4 * L)
        for u in range(4):
            off = b + u * L
            idxs = off + lanes
            v0 = sel_v[0, pl.ds(off, L)]
            v1 = sel_v[1, pl.ds(off, L)]
            t0 = v0 > bv0
            t1 = v1 > bv1
            bv0 = jnp.where(t0, v0, bv0)
            bi0 = jnp.where(t0, idxs, bi0)
            bv1 = jnp.where(t1, v1, bv1)
            bi1 = jnp.where(t1, idxs, bi1)
        return bv0, bi0, bv1, bi1

    bv0, bi0, bv1, bi1 = lax.fori_loop(
        0, N_RES // (4 * L), body, (ninf, zero, ninf, zero))

    for shift in (8, 4, 2, 1):
        perm = lanes ^ shift
        ov0, oi0 = _lane_perm(bv0, perm), _lane_perm(bi0, perm)
        ov1, oi1 = _lane_perm(bv1, perm), _lane_perm(bi1, perm)
        b0 = (ov0 > bv0) | ((ov0 == bv0) & (oi0 < bi0))
        b1 = (ov1 > bv1) | ((ov1 == bv1) & (oi1 < bi1))
        bv0 = jnp.where(b0, ov0, bv0)
        bi0 = jnp.where(b0, oi0, bi0)
        bv1 = jnp.where(b1, ov1, bv1)
        bi1 = jnp.where(b1, oi1, bi1)

    iv = jnp.where(lanes == 1, bi1, bi0)
    idx_v[...] = iv

    pltpu.async_copy(items_hbm.at[idx_v.at[pl.ds(0, ROWS_PER_W)]],
                     rows_v, sem).wait()
    pltpu.sync_copy(rows_v, out_hbm.at[pl.ds(base, ROWS_PER_W)])


def _select_gather(sel2d, items):
    mesh = plsc.VectorSubcoreMesh(core_axis_name="c", subcore_axis_name="s")
    return pl.kernel(
        _sc_select_gather,
        mesh=mesh,
        out_type=jax.ShapeDtypeStruct((N_ROWS, N_RES), jnp.float32),
        scratch_types=[
            pltpu.VMEM((ROWS_PER_W, N_RES), jnp.float32),
            pltpu.VMEM((L,), jnp.int32),
            pltpu.VMEM((ROWS_PER_W, N_RES), jnp.float32),
            pltpu.SemaphoreType.DMA,
        ],
    )(sel2d, items)


NT = 1024


def _mm_body(g_ref, w_ref, o_ref):
    part = jnp.dot(g_ref[...], w_ref[...],
                   preferred_element_type=jnp.float32)
    for e in range(N_EVENTS):
        o_ref[0, e] = lax.slice_in_dim(part, INSTR * e, INSTR * (e + 1),
                                       axis=0)


def _postprocess(gathered, waves):
    return pl.pallas_call(
        _mm_body,
        grid=(N_SAMPLES // NT,),
        in_specs=[
            pl.BlockSpec((N_ROWS, N_RES), lambda j: (0, 0)),
            pl.BlockSpec((N_RES, NT), lambda j: (0, j)),
        ],
        out_specs=pl.BlockSpec((1, N_EVENTS, INSTR, NT),
                               lambda j: (0, 0, 0, j)),
        out_shape=jax.ShapeDtypeStruct((1, N_EVENTS, INSTR, N_SAMPLES),
                                       jnp.float32),
    )(gathered, waves)


def kernel(selections, items, waves):
    b, e, x, n = selections.shape
    sel2d = selections.reshape(b * e * x, n)
    gathered = _select_gather(sel2d, items)
    return _postprocess(gathered, waves)
